# baseline (device time: 22705 ns/iter reference)
import jax
import jax.numpy as jnp
from jax import lax
from jax.experimental import pallas as pl
from jax.experimental.pallas import tpu as pltpu

N_Y = 4
SUB = 4


def kernel(dy, W):
    m, k = dy.shape
    d = W.shape[0]
    ch = m // N_Y
    sb = ch // SUB

    def body(
        dy_ref,
        w_ref,
        out_ref,
        wbf_ref,
        psend_ref,
        rs_buf,
        rs_send_sems,
        rs_recv_sems,
        ag_send_sems,
        ag_recv_sems,
    ):
        my_x = lax.axis_index("x")
        my_y = lax.axis_index("y")
        my_z = lax.axis_index("z")

        barrier_sem = pltpu.get_barrier_semaphore()
        for off in range(1, N_Y):
            peer = (my_y + off) % N_Y
            pl.semaphore_signal(
                barrier_sem,
                inc=1,
                device_id=(my_x, peer, my_z),
                device_id_type=pl.DeviceIdType.MESH,
            )

        wbf_ref[:, :] = w_ref[:, :].astype(jnp.bfloat16)
        dnums = (((1,), (1,)), ((), ()))

        rs_sends = []
        for off in range(1, N_Y):
            dst = (my_y + off) % N_Y
            slot = off - 1
            a = dy_ref[pl.ds(dst * ch, ch), :].astype(jnp.bfloat16)
            pchunk = lax.dot_general(
                a, wbf_ref[:, :], dnums, preferred_element_type=jnp.float32
            )
            psend_ref[slot, :, :] = pchunk.astype(jnp.bfloat16)
            if off == 1:
                pl.semaphore_wait(barrier_sem, N_Y - 1)
            for s in range(SUB):
                rdma = pltpu.make_async_remote_copy(
                    src_ref=psend_ref.at[slot, pl.ds(s * sb, sb), :],
                    dst_ref=rs_buf.at[slot, pl.ds(s * sb, sb), :],
                    send_sem=rs_send_sems.at[slot, s],
                    recv_sem=rs_recv_sems.at[slot, s],
                    device_id=(my_x, dst, my_z),
                    device_id_type=pl.DeviceIdType.MESH,
                )
                rdma.start()
                rs_sends.append(rdma)

        a_own = dy_ref[pl.ds(my_y * ch, ch), :].astype(jnp.bfloat16)
        own = lax.dot_general(
            a_own, wbf_ref[:, :], dnums, preferred_element_type=jnp.float32
        )

        ag_sends = []
        for s in range(SUB):
            for slot in range(N_Y - 1):
                rs_sends[slot * SUB + s].wait_recv()
            sub_red = own[s * sb : (s + 1) * sb, :]
            for slot in range(N_Y - 1):
                sub_red = sub_red + rs_buf[
                    slot, pl.ds(s * sb, sb), :
                ].astype(jnp.float32)
            row0 = my_y * ch + s * sb
            out_ref[pl.ds(row0, sb), :] = sub_red.astype(jnp.bfloat16)
            for off in range(1, N_Y):
                dst = (my_y + off) % N_Y
                slot = off - 1
                rdma = pltpu.make_async_remote_copy(
                    src_ref=out_ref.at[pl.ds(row0, sb), :],
                    dst_ref=out_ref.at[pl.ds(row0, sb), :],
                    send_sem=ag_send_sems.at[slot, s],
                    recv_sem=ag_recv_sems.at[slot, s],
                    device_id=(my_x, dst, my_z),
                    device_id_type=pl.DeviceIdType.MESH,
                )
                rdma.start()
                ag_sends.append(rdma)

        for rdma in ag_sends:
            rdma.wait_recv()
        for rdma in rs_sends:
            rdma.wait_send()
        for rdma in ag_sends:
            rdma.wait_send()

    return pl.pallas_call(
        body,
        out_shape=jax.ShapeDtypeStruct((m, d), jnp.bfloat16),
        in_specs=[
            pl.BlockSpec(memory_space=pltpu.VMEM),
            pl.BlockSpec(memory_space=pltpu.VMEM),
        ],
        out_specs=pl.BlockSpec(memory_space=pltpu.VMEM),
        scratch_shapes=[
            pltpu.VMEM((d, k), jnp.bfloat16),
            pltpu.VMEM((N_Y - 1, ch, d), jnp.bfloat16),
            pltpu.VMEM((N_Y - 1, ch, d), jnp.bfloat16),
            pltpu.SemaphoreType.DMA((N_Y - 1, SUB)),
            pltpu.SemaphoreType.DMA((N_Y - 1, SUB)),
            pltpu.SemaphoreType.DMA((N_Y - 1, SUB)),
            pltpu.SemaphoreType.DMA((N_Y - 1, SUB)),
        ],
        compiler_params=pltpu.CompilerParams(collective_id=0),
    )(dy, W)


# device time: 21586 ns/iter; 1.0518x vs baseline; 1.0518x over previous
import jax
import jax.numpy as jnp
from jax import lax
from jax.experimental import pallas as pl
from jax.experimental.pallas import tpu as pltpu

N_Y = 4
N_Z = 4


def kernel(dy, W):
    m, k = dy.shape
    d = W.shape[0]
    ch = m // N_Z
    sb = ch // N_Y

    def body(
        dy_ref,
        w_ref,
        out_ref,
        wbf_ref,
        pband_ref,
        psend_ref,
        rs_buf,
        rs_send_sems,
        rs_recv_sems,
        yag_send_sems,
        yag_recv_sems,
        zag_send_sems,
        zag_recv_sems,
    ):
        my_x = lax.axis_index("x")
        my_y = lax.axis_index("y")
        my_z = lax.axis_index("z")

        barrier_sem = pltpu.get_barrier_semaphore()
        for off in range(1, N_Y):
            pl.semaphore_signal(
                barrier_sem, inc=1,
                device_id=(my_x, (my_y + off) % N_Y, my_z),
                device_id_type=pl.DeviceIdType.MESH,
            )
        for off in range(1, N_Z):
            pl.semaphore_signal(
                barrier_sem, inc=1,
                device_id=(my_x, my_y, (my_z + off) % N_Z),
                device_id_type=pl.DeviceIdType.MESH,
            )

        wbf_ref[:, :] = w_ref[:, :].astype(jnp.bfloat16)
        a = dy_ref[pl.ds(my_z * ch, ch), :].astype(jnp.bfloat16)
        pband_ref[:, :] = lax.dot_general(
            a, wbf_ref[:, :], (((1,), (1,)), ((), ())),
            preferred_element_type=jnp.float32,
        )
        pl.semaphore_wait(barrier_sem, (N_Y - 1) + (N_Z - 1))

        rs_sends = []
        for off in range(1, N_Y):
            j = (my_y + off) % N_Y
            slot = off - 1
            psend_ref[slot, :, :] = pband_ref[
                pl.ds(j * sb, sb), :
            ].astype(jnp.bfloat16)
            rdma = pltpu.make_async_remote_copy(
                src_ref=psend_ref.at[slot],
                dst_ref=rs_buf.at[slot],
                send_sem=rs_send_sems.at[slot],
                recv_sem=rs_recv_sems.at[slot],
                device_id=(my_x, j, my_z),
                device_id_type=pl.DeviceIdType.MESH,
            )
            rdma.start()
            rs_sends.append(rdma)

        red = pband_ref[pl.ds(my_y * sb, sb), :]
        for slot in range(N_Y - 1):
            rs_sends[slot].wait_recv()
            red = red + rs_buf[slot, :, :].astype(jnp.float32)
        row_own = my_z * ch + my_y * sb
        out_ref[pl.ds(row_own, sb), :] = red.astype(jnp.bfloat16)

        yag_sends = []
        for off in range(1, N_Y):
            j = (my_y + off) % N_Y
            slot = off - 1
            rdma = pltpu.make_async_remote_copy(
                src_ref=out_ref.at[pl.ds(row_own, sb), :],
                dst_ref=out_ref.at[pl.ds(row_own, sb), :],
                send_sem=yag_send_sems.at[slot],
                recv_sem=yag_recv_sems.at[slot],
                device_id=(my_x, j, my_z),
                device_id_type=pl.DeviceIdType.MESH,
            )
            rdma.start()
            yag_sends.append(rdma)

        zag_sends = []
        for t in range(N_Y):
            if t == 0:
                row_t = row_own
            else:
                jsrc = (my_y - t) % N_Y
                row_t = my_z * ch + jsrc * sb
                recv = pltpu.make_async_remote_copy(
                    src_ref=out_ref.at[pl.ds(row_t, sb), :],
                    dst_ref=out_ref.at[pl.ds(row_t, sb), :],
                    send_sem=yag_send_sems.at[t - 1],
                    recv_sem=yag_recv_sems.at[t - 1],
                    device_id=(my_x, my_y, my_z),
                    device_id_type=pl.DeviceIdType.MESH,
                )
                recv.wait_recv()
            for off in range(1, N_Z):
                dz = (my_z + off) % N_Z
                rdma = pltpu.make_async_remote_copy(
                    src_ref=out_ref.at[pl.ds(row_t, sb), :],
                    dst_ref=out_ref.at[pl.ds(row_t, sb), :],
                    send_sem=zag_send_sems.at[off - 1, t],
                    recv_sem=zag_recv_sems.at[off - 1, t],
                    device_id=(my_x, my_y, dz),
                    device_id_type=pl.DeviceIdType.MESH,
                )
                rdma.start()
                zag_sends.append(rdma)

        for sl in range(N_Z - 1):
            for t in range(N_Y):
                recv = pltpu.make_async_remote_copy(
                    src_ref=out_ref.at[pl.ds(0, sb), :],
                    dst_ref=out_ref.at[pl.ds(0, sb), :],
                    send_sem=zag_send_sems.at[sl, t],
                    recv_sem=zag_recv_sems.at[sl, t],
                    device_id=(my_x, my_y, my_z),
                    device_id_type=pl.DeviceIdType.MESH,
                )
                recv.wait_recv()

        for rdma in rs_sends:
            rdma.wait_send()
        for rdma in yag_sends:
            rdma.wait_send()
        for rdma in zag_sends:
            rdma.wait_send()

    return pl.pallas_call(
        body,
        out_shape=jax.ShapeDtypeStruct((m, d), jnp.bfloat16),
        in_specs=[
            pl.BlockSpec(memory_space=pltpu.VMEM),
            pl.BlockSpec(memory_space=pltpu.VMEM),
        ],
        out_specs=pl.BlockSpec(memory_space=pltpu.VMEM),
        scratch_shapes=[
            pltpu.VMEM((d, k), jnp.bfloat16),
            pltpu.VMEM((ch, d), jnp.float32),
            pltpu.VMEM((N_Y - 1, sb, d), jnp.bfloat16),
            pltpu.VMEM((N_Y - 1, sb, d), jnp.bfloat16),
            pltpu.SemaphoreType.DMA((N_Y - 1,)),
            pltpu.SemaphoreType.DMA((N_Y - 1,)),
            pltpu.SemaphoreType.DMA((N_Y - 1,)),
            pltpu.SemaphoreType.DMA((N_Y - 1,)),
            pltpu.SemaphoreType.DMA((N_Z - 1, N_Y)),
            pltpu.SemaphoreType.DMA((N_Z - 1, N_Y)),
        ],
        compiler_params=pltpu.CompilerParams(collective_id=0),
    )(dy, W)
